# Initial kernel scaffold; baseline (speedup 1.0000x reference)
import jax
import jax.numpy as jnp
from jax.experimental import pallas as pl


def kernel(x_enc, Wemb, Wqk, Wv, Wout, bout, g1, b1, g2, b2, Wc1, bc1, Wc2, bc2, rotations, lnf_g, lnf_b, Wproj, bproj):
    raise NotImplementedError("write your pallas kernel here")


# speedup vs baseline: 1.0406x; 1.0406x over previous
import functools

import numpy as np
import jax
import jax.numpy as jnp
from jax.experimental import pallas as pl
from jax.experimental.pallas import tpu as pltpu

B, L, PRED = 8, 2048, 336
D, H, E = 128, 8, 2
DH = D // H
BUCKET, NHASH = 32, 4
NBUCK = L // BUCKET
BH = B * H
NCH = NHASH * NBUCK
TOT = NHASH * L


def _bdot(a, b):
    return jnp.dot(a.astype(jnp.bfloat16), b.astype(jnp.bfloat16),
                   preferred_element_type=jnp.float32)


def _positional_embedding():
    pos = np.arange(L, dtype=np.float32)[:, None]
    div = np.exp(np.arange(0, D, 2, dtype=np.float32) * -(np.log(10000.0) / D))
    pe = np.zeros((L, D), dtype=np.float32)
    pe[:, 0::2] = np.sin(pos * div)
    pe[:, 1::2] = np.cos(pos * div)
    return jnp.asarray(pe)


def _embed_body(win_ref, pe_ref, w3_ref, out_ref):
    win = win_ref[0]
    w3 = w3_ref[...]
    f32 = jnp.float32
    bf = lambda t: t.astype(jnp.bfloat16).astype(f32)
    acc = bf(win[:, 0:1]) * bf(w3[0:1, :])
    acc = acc + bf(win[:, 1:2]) * bf(w3[1:2, :])
    acc = acc + bf(win[:, 2:3]) * bf(w3[2:3, :])
    out_ref[0] = acc + pe_ref[...]


def _embed(win, pe, w3):
    return pl.pallas_call(
        _embed_body,
        grid=(B,),
        in_specs=[
            pl.BlockSpec((1, L, 3), lambda b: (b, 0, 0)),
            pl.BlockSpec((L, D), lambda b: (0, 0)),
            pl.BlockSpec((8, D), lambda b: (0, 0)),
        ],
        out_specs=pl.BlockSpec((1, L, D), lambda b: (b, 0, 0)),
        out_shape=jax.ShapeDtypeStruct((B, L, D), jnp.float32),
    )(win, pe, w3)


LBP = 512


def _proj_body(h_ref, wqk_ref, wv_ref, rot_ref, qk_ref, v_ref, bk_ref):
    x = h_ref[0]
    wqk = wqk_ref[...]
    wv = wv_ref[...]
    rot = rot_ref[...]
    qk = _bdot(x, wqk)
    v = _bdot(x, wv)
    qk_ref[0] = qk
    v_ref[0] = v
    for hh in range(H):
        qh = qk[:, hh * DH:(hh + 1) * DH]
        ra = _bdot(qh, rot)
        for r in range(NHASH):
            rr = ra[:, r * (NBUCK // 2):(r + 1) * (NBUCK // 2)]
            full = jnp.concatenate([rr, -rr], axis=-1)
            am = jnp.argmax(full, axis=-1).astype(jnp.int32)
            bk_ref[0, hh, r] = am


def _proj(h, wqk, wv, rot):
    return pl.pallas_call(
        _proj_body,
        grid=(B, L // LBP),
        in_specs=[
            pl.BlockSpec((1, LBP, D), lambda b, l: (b, l, 0)),
            pl.BlockSpec((D, D), lambda b, l: (0, 0)),
            pl.BlockSpec((D, D), lambda b, l: (0, 0)),
            pl.BlockSpec((DH, NHASH * (NBUCK // 2)), lambda b, l: (0, 0)),
        ],
        out_specs=[
            pl.BlockSpec((1, LBP, D), lambda b, l: (b, l, 0)),
            pl.BlockSpec((1, LBP, D), lambda b, l: (b, l, 0)),
            pl.BlockSpec((1, H, NHASH, LBP), lambda b, l: (b, 0, 0, l)),
        ],
        out_shape=[
            jax.ShapeDtypeStruct((B, L, D), jnp.float32),
            jax.ShapeDtypeStruct((B, L, D), jnp.float32),
            jax.ShapeDtypeStruct((B, H, NHASH, L), jnp.int32),
        ],
    )(h, wqk, wv, rot)


CG = 32
NG = NCH // CG


def _attn_body(sqk_ref, sv_ref, st_ref, so_ref, slog_ref):
    sqk = sqk_ref[0]
    sv = sv_ref[0]
    stv = st_ref[0]
    nrm = jnp.sqrt(jnp.sum(sqk * sqk, axis=-1, keepdims=True))
    kn = sqk / jnp.maximum(nrm, 1e-12)
    knr = jnp.concatenate([kn[-BUCKET:], kn[:-BUCKET]], axis=0)
    svr = jnp.concatenate([sv[-BUCKET:], sv[:-BUCKET]], axis=0)
    str_ = jnp.concatenate([stv[-1:], stv[:-1]], axis=0)
    scale = DH ** -0.5

    for g in range(NG):
        s0 = g * CG * BUCKET
        q = sqk[s0:s0 + CG * BUCKET].reshape(CG, BUCKET, DH)
        kc = kn[s0:s0 + CG * BUCKET].reshape(CG, BUCKET, DH)
        kp = knr[s0:s0 + CG * BUCKET].reshape(CG, BUCKET, DH)
        vc = sv[s0:s0 + CG * BUCKET].reshape(CG, BUCKET, DH)
        vp = svr[s0:s0 + CG * BUCKET].reshape(CG, BUCKET, DH)
        tq = stv[g * CG:(g + 1) * CG]
        tp = str_[g * CG:(g + 1) * CG]
        bk = jnp.concatenate([kc, kp], axis=1)
        bv = jnp.concatenate([vc, vp], axis=1)
        bt = jnp.concatenate([tq, tp], axis=1)
        dots = jax.lax.dot_general(
            q.astype(jnp.bfloat16), bk.astype(jnp.bfloat16),
            (((2,), (2,)), ((0,), (0,))),
            preferred_element_type=jnp.float32) * scale
        dots = jnp.where(tq[:, :, None] == bt[:, None, :], -5e4, dots)
        mx = jnp.max(dots, axis=-1, keepdims=True)
        ex = jnp.exp(dots - mx)
        sm = jnp.sum(ex, axis=-1, keepdims=True)
        lse = mx + jnp.log(sm)
        probs = ex / sm
        bo = jax.lax.dot_general(
            probs.astype(jnp.bfloat16), bv.astype(jnp.bfloat16),
            (((2,), (1,)), ((0,), (0,))),
            preferred_element_type=jnp.float32)
        so_ref[0, s0:s0 + CG * BUCKET] = bo.reshape(CG * BUCKET, DH)
        slog_ref[0, g * CG:(g + 1) * CG] = lse[:, :, 0]


def _attn(sqk, sv, st):
    return pl.pallas_call(
        _attn_body,
        grid=(BH,),
        in_specs=[
            pl.BlockSpec((1, TOT, DH), lambda i: (i, 0, 0)),
            pl.BlockSpec((1, TOT, DH), lambda i: (i, 0, 0)),
            pl.BlockSpec((1, NCH, BUCKET), lambda i: (i, 0, 0)),
        ],
        out_specs=[
            pl.BlockSpec((1, TOT, DH), lambda i: (i, 0, 0)),
            pl.BlockSpec((1, NCH, BUCKET), lambda i: (i, 0, 0)),
        ],
        out_shape=[
            jax.ShapeDtypeStruct((BH, TOT, DH), jnp.float32),
            jax.ShapeDtypeStruct((BH, NCH, BUCKET), jnp.float32),
        ],
    )(sqk, sv, st)


LBC = 512


def _combine_body(o_ref, lg_ref, h_ref, wout_ref, bout_ref, g1_ref, b1_ref,
                  g2_ref, b2_ref, wc1_ref, bc1_ref, wc2_ref, bc2_ref, out_ref):
    lg = lg_ref[0]
    o = o_ref[0]
    mx = jnp.max(lg, axis=1, keepdims=True)
    w = jnp.exp(lg - mx)
    w = w / jnp.sum(w, axis=1, keepdims=True)
    att = jnp.sum(o * w[..., None], axis=1)
    wout = wout_ref[...]
    acc = jnp.zeros((LBC, D), jnp.float32)
    for hh in range(H):
        acc = acc + _bdot(att[hh], wout[hh * DH:(hh + 1) * DH, :])
    x = h_ref[0] + acc + bout_ref[...][0:1, :]
    m = jnp.mean(x, axis=-1, keepdims=True)
    v = jnp.mean((x - m) ** 2, axis=-1, keepdims=True)
    x = (x - m) / jnp.sqrt(v + 1e-5) * g1_ref[...][0:1, :] + b1_ref[...][0:1, :]
    y = _bdot(x, wc1_ref[...]) + bc1_ref[...][0:1, :]
    y = 0.5 * y * (1.0 + jax.lax.erf(y * (2.0 ** -0.5)))
    y = _bdot(y, wc2_ref[...]) + bc2_ref[...][0:1, :]
    z = x + y
    m = jnp.mean(z, axis=-1, keepdims=True)
    v = jnp.mean((z - m) ** 2, axis=-1, keepdims=True)
    out_ref[0] = (z - m) / jnp.sqrt(v + 1e-5) * g2_ref[...][0:1, :] \
        + b2_ref[...][0:1, :]


def _combine(o5, lg5, h, wout, bout, g1, b1, g2, b2, wc1, bc1, wc2, bc2):
    row = lambda t: t.reshape(1, D)
    return pl.pallas_call(
        _combine_body,
        grid=(B, L // LBC),
        in_specs=[
            pl.BlockSpec((1, H, NHASH, LBC, DH), lambda b, l: (b, 0, 0, l, 0)),
            pl.BlockSpec((1, H, NHASH, LBC), lambda b, l: (b, 0, 0, l)),
            pl.BlockSpec((1, LBC, D), lambda b, l: (b, l, 0)),
            pl.BlockSpec((D, D), lambda b, l: (0, 0)),
            pl.BlockSpec((1, D), lambda b, l: (0, 0)),
            pl.BlockSpec((1, D), lambda b, l: (0, 0)),
            pl.BlockSpec((1, D), lambda b, l: (0, 0)),
            pl.BlockSpec((1, D), lambda b, l: (0, 0)),
            pl.BlockSpec((1, D), lambda b, l: (0, 0)),
            pl.BlockSpec((D, D), lambda b, l: (0, 0)),
            pl.BlockSpec((1, D), lambda b, l: (0, 0)),
            pl.BlockSpec((D, D), lambda b, l: (0, 0)),
            pl.BlockSpec((1, D), lambda b, l: (0, 0)),
        ],
        out_specs=pl.BlockSpec((1, LBC, D), lambda b, l: (b, l, 0)),
        out_shape=jax.ShapeDtypeStruct((B, L, D), jnp.float32),
    )(o5, lg5, h, wout, row(bout), row(g1), row(b1), row(g2), row(b2),
      wc1, row(bc1), wc2, row(bc2))


def _final_body(h_ref, g_ref, b_ref, wp_ref, bp_ref, out_ref):
    x = h_ref[0]
    m = jnp.mean(x, axis=-1, keepdims=True)
    v = jnp.mean((x - m) ** 2, axis=-1, keepdims=True)
    x = (x - m) / jnp.sqrt(v + 1e-5) * g_ref[...][0:1, :] + b_ref[...][0:1, :]
    bf = lambda t: t.astype(jnp.bfloat16).astype(jnp.float32)
    y = jnp.sum(bf(x) * bf(wp_ref[...][0:1, :]), axis=-1) + bp_ref[0, 0]
    out_ref[0, 0] = y


def _final(htail, lnf_g, lnf_b, wproj, bproj):
    out = pl.pallas_call(
        _final_body,
        grid=(B,),
        in_specs=[
            pl.BlockSpec((1, PRED, D), lambda b: (b, 0, 0)),
            pl.BlockSpec((1, D), lambda b: (0, 0)),
            pl.BlockSpec((1, D), lambda b: (0, 0)),
            pl.BlockSpec((1, D), lambda b: (0, 0)),
            pl.BlockSpec((1, 1), lambda b: (0, 0)),
        ],
        out_specs=pl.BlockSpec((1, 1, PRED), lambda b: (b, 0, 0)),
        out_shape=jax.ShapeDtypeStruct((B, 1, PRED), jnp.float32),
    )(htail, lnf_g.reshape(1, D), lnf_b.reshape(1, D),
      wproj.reshape(1, D), bproj.reshape(1, 1))
    return out[:, 0, :]


def _route(qk, v, bkts):
    qkh = qk.reshape(B, L, H, DH).transpose(0, 2, 1, 3).reshape(BH, L, DH)
    vh = v.reshape(B, L, H, DH).transpose(0, 2, 1, 3).reshape(BH, L, DH)
    bc = bkts.reshape(BH, NHASH, L) + (jnp.arange(NHASH) * NBUCK)[None, :, None]
    bc = bc.reshape(BH, TOT)
    ticker = jnp.arange(TOT)[None, :]
    bt = L * bc + (ticker % L)
    sticker = jnp.argsort(bt, axis=-1)
    undo = jnp.argsort(sticker, axis=-1)
    st = sticker % L
    sqk = jnp.take_along_axis(qkh, st[..., None], axis=1)
    sv = jnp.take_along_axis(vh, st[..., None], axis=1)
    return sqk, sv, st, undo


def kernel(x_enc, Wemb, Wqk, Wv, Wout, bout, g1, b1, g2, b2, Wc1, bc1,
           Wc2, bc2, rotations, lnf_g, lnf_b, Wproj, bproj):
    pe = _positional_embedding()
    xp = jnp.concatenate([x_enc[:, -1:], x_enc, x_enc[:, :1]], axis=1)
    win = jnp.stack([xp[:, :-2], xp[:, 1:-1], xp[:, 2:]], axis=-1)
    w3 = jnp.zeros((8, D), jnp.float32).at[:3, :].set(Wemb[:, 0, :].T)
    h = _embed(win, pe, w3)

    for i in range(E):
        rot = rotations[i].reshape(DH, NHASH * (NBUCK // 2))
        qk, v, bkts = _proj(h, Wqk[i], Wv[i], rot)
        sqk, sv, st, undo = _route(qk, v, bkts)
        st3 = st.reshape(BH, NCH, BUCKET)
        so, slog = _attn(sqk, sv, st3)
        slog = slog.reshape(BH, TOT)
        o = jnp.take_along_axis(so, undo[..., None], axis=1)
        lg = jnp.take_along_axis(slog, undo, axis=1)
        o5 = o.reshape(B, H, NHASH, L, DH)
        lg5 = lg.reshape(B, H, NHASH, L)
        h = _combine(o5, lg5, h, Wout[i], bout[i], g1[i], b1[i], g2[i],
                     b2[i], Wc1[i], bc1[i], Wc2[i], bc2[i])

    htail = h[:, -PRED:, :]
    return _final(htail, lnf_g, lnf_b, Wproj, bproj)


# speedup vs baseline: 1.0617x; 1.0203x over previous
import functools

import numpy as np
import jax
import jax.numpy as jnp
from jax import lax
from jax.experimental import pallas as pl
from jax.experimental.pallas import tpu as pltpu
from jax.experimental.pallas import tpu_sc as plsc

B, L, PRED = 8, 2048, 336
D, H, E = 128, 8, 2
DH = D // H
BUCKET, NHASH = 32, 4
NBUCK = L // BUCKET
BH = B * H
NCH = NHASH * NBUCK
TOT = NHASH * L


def _bdot(a, b):
    return jnp.dot(a.astype(jnp.bfloat16), b.astype(jnp.bfloat16),
                   preferred_element_type=jnp.float32)


def _positional_embedding():
    pos = np.arange(L, dtype=np.float32)[:, None]
    div = np.exp(np.arange(0, D, 2, dtype=np.float32) * -(np.log(10000.0) / D))
    pe = np.zeros((L, D), dtype=np.float32)
    pe[:, 0::2] = np.sin(pos * div)
    pe[:, 1::2] = np.cos(pos * div)
    return jnp.asarray(pe)


def _embed_body(win_ref, pe_ref, w3_ref, out_ref):
    win = win_ref[0]
    w3 = w3_ref[...]
    f32 = jnp.float32
    bf = lambda t: t.astype(jnp.bfloat16).astype(f32)
    acc = bf(win[:, 0:1]) * bf(w3[0:1, :])
    acc = acc + bf(win[:, 1:2]) * bf(w3[1:2, :])
    acc = acc + bf(win[:, 2:3]) * bf(w3[2:3, :])
    out_ref[0] = acc + pe_ref[...]


def _embed(win, pe, w3):
    return pl.pallas_call(
        _embed_body,
        grid=(B,),
        in_specs=[
            pl.BlockSpec((1, L, 3), lambda b: (b, 0, 0)),
            pl.BlockSpec((L, D), lambda b: (0, 0)),
            pl.BlockSpec((8, D), lambda b: (0, 0)),
        ],
        out_specs=pl.BlockSpec((1, L, D), lambda b: (b, 0, 0)),
        out_shape=jax.ShapeDtypeStruct((B, L, D), jnp.float32),
    )(win, pe, w3)


LBP = 512


def _proj_body(h_ref, wqk_ref, wv_ref, rot_ref, qk_ref, v_ref, bk_ref):
    x = h_ref[0]
    wqk = wqk_ref[...]
    wv = wv_ref[...]
    rot = rot_ref[...]
    qk = _bdot(x, wqk)
    v = _bdot(x, wv)
    qk_ref[0] = qk
    v_ref[0] = v
    for hh in range(H):
        qh = qk[:, hh * DH:(hh + 1) * DH]
        ra = _bdot(qh, rot)
        for r in range(NHASH):
            rr = ra[:, r * (NBUCK // 2):(r + 1) * (NBUCK // 2)]
            full = jnp.concatenate([rr, -rr], axis=-1)
            am = jnp.argmax(full, axis=-1).astype(jnp.int32)
            bk_ref[0, hh, r] = am


def _proj(h, wqk, wv, rot):
    return pl.pallas_call(
        _proj_body,
        grid=(B, L // LBP),
        in_specs=[
            pl.BlockSpec((1, LBP, D), lambda b, l: (b, l, 0)),
            pl.BlockSpec((D, D), lambda b, l: (0, 0)),
            pl.BlockSpec((D, D), lambda b, l: (0, 0)),
            pl.BlockSpec((DH, NHASH * (NBUCK // 2)), lambda b, l: (0, 0)),
        ],
        out_specs=[
            pl.BlockSpec((1, LBP, D), lambda b, l: (b, l, 0)),
            pl.BlockSpec((1, LBP, D), lambda b, l: (b, l, 0)),
            pl.BlockSpec((1, H, NHASH, LBP), lambda b, l: (b, 0, 0, l)),
        ],
        out_shape=[
            jax.ShapeDtypeStruct((B, L, D), jnp.float32),
            jax.ShapeDtypeStruct((B, L, D), jnp.float32),
            jax.ShapeDtypeStruct((B, H, NHASH, L), jnp.int32),
        ],
    )(h, wqk, wv, rot)


CG = 32
NG = NCH // CG


def _attn_body(sqk_ref, sv_ref, st_ref, so_ref, slog_ref):
    sqk = sqk_ref[0]
    sv = sv_ref[0]
    stv = st_ref[0]
    nrm = jnp.sqrt(jnp.sum(sqk * sqk, axis=-1, keepdims=True))
    kn = sqk / jnp.maximum(nrm, 1e-12)
    knr = jnp.concatenate([kn[-BUCKET:], kn[:-BUCKET]], axis=0)
    svr = jnp.concatenate([sv[-BUCKET:], sv[:-BUCKET]], axis=0)
    str_ = jnp.concatenate([stv[-1:], stv[:-1]], axis=0)
    scale = DH ** -0.5

    for g in range(NG):
        s0 = g * CG * BUCKET
        q = sqk[s0:s0 + CG * BUCKET].reshape(CG, BUCKET, DH)
        kc = kn[s0:s0 + CG * BUCKET].reshape(CG, BUCKET, DH)
        kp = knr[s0:s0 + CG * BUCKET].reshape(CG, BUCKET, DH)
        vc = sv[s0:s0 + CG * BUCKET].reshape(CG, BUCKET, DH)
        vp = svr[s0:s0 + CG * BUCKET].reshape(CG, BUCKET, DH)
        tq = stv[g * CG:(g + 1) * CG]
        tp = str_[g * CG:(g + 1) * CG]
        bk = jnp.concatenate([kc, kp], axis=1)
        bv = jnp.concatenate([vc, vp], axis=1)
        bt = jnp.concatenate([tq, tp], axis=1)
        dots = jax.lax.dot_general(
            q.astype(jnp.bfloat16), bk.astype(jnp.bfloat16),
            (((2,), (2,)), ((0,), (0,))),
            preferred_element_type=jnp.float32) * scale
        dots = jnp.where(tq[:, :, None] == bt[:, None, :], -5e4, dots)
        mx = jnp.max(dots, axis=-1, keepdims=True)
        ex = jnp.exp(dots - mx)
        sm = jnp.sum(ex, axis=-1, keepdims=True)
        lse = mx + jnp.log(sm)
        probs = ex / sm
        bo = jax.lax.dot_general(
            probs.astype(jnp.bfloat16), bv.astype(jnp.bfloat16),
            (((2,), (1,)), ((0,), (0,))),
            preferred_element_type=jnp.float32)
        so_ref[0, s0:s0 + CG * BUCKET] = bo.reshape(CG * BUCKET, DH)
        slog_ref[0, g * CG:(g + 1) * CG] = lse[:, :, 0]


def _attn(sqk, sv, st):
    return pl.pallas_call(
        _attn_body,
        grid=(BH,),
        in_specs=[
            pl.BlockSpec((1, TOT, DH), lambda i: (i, 0, 0)),
            pl.BlockSpec((1, TOT, DH), lambda i: (i, 0, 0)),
            pl.BlockSpec((1, NCH, BUCKET), lambda i: (i, 0, 0)),
        ],
        out_specs=[
            pl.BlockSpec((1, TOT, DH), lambda i: (i, 0, 0)),
            pl.BlockSpec((1, NCH, BUCKET), lambda i: (i, 0, 0)),
        ],
        out_shape=[
            jax.ShapeDtypeStruct((BH, TOT, DH), jnp.float32),
            jax.ShapeDtypeStruct((BH, NCH, BUCKET), jnp.float32),
        ],
    )(sqk, sv, st)


LBC = 512


def _combine_body(o_ref, lg_ref, h_ref, wout_ref, bout_ref, g1_ref, b1_ref,
                  g2_ref, b2_ref, wc1_ref, bc1_ref, wc2_ref, bc2_ref, out_ref):
    lg = lg_ref[0]
    o = o_ref[0]
    mx = jnp.max(lg, axis=1, keepdims=True)
    w = jnp.exp(lg - mx)
    w = w / jnp.sum(w, axis=1, keepdims=True)
    att = jnp.sum(o * w[..., None], axis=1)
    wout = wout_ref[...]
    acc = jnp.zeros((LBC, D), jnp.float32)
    for hh in range(H):
        acc = acc + _bdot(att[hh], wout[hh * DH:(hh + 1) * DH, :])
    x = h_ref[0] + acc + bout_ref[...][0:1, :]
    m = jnp.mean(x, axis=-1, keepdims=True)
    v = jnp.mean((x - m) ** 2, axis=-1, keepdims=True)
    x = (x - m) / jnp.sqrt(v + 1e-5) * g1_ref[...][0:1, :] + b1_ref[...][0:1, :]
    y = _bdot(x, wc1_ref[...]) + bc1_ref[...][0:1, :]
    y = 0.5 * y * (1.0 + jax.lax.erf(y * (2.0 ** -0.5)))
    y = _bdot(y, wc2_ref[...]) + bc2_ref[...][0:1, :]
    z = x + y
    m = jnp.mean(z, axis=-1, keepdims=True)
    v = jnp.mean((z - m) ** 2, axis=-1, keepdims=True)
    out_ref[0] = (z - m) / jnp.sqrt(v + 1e-5) * g2_ref[...][0:1, :] \
        + b2_ref[...][0:1, :]


def _combine(o5, lg5, h, wout, bout, g1, b1, g2, b2, wc1, bc1, wc2, bc2):
    row = lambda t: t.reshape(1, D)
    return pl.pallas_call(
        _combine_body,
        grid=(B, L // LBC),
        in_specs=[
            pl.BlockSpec((1, H, NHASH, LBC, DH), lambda b, l: (b, 0, 0, l, 0)),
            pl.BlockSpec((1, H, NHASH, LBC), lambda b, l: (b, 0, 0, l)),
            pl.BlockSpec((1, LBC, D), lambda b, l: (b, l, 0)),
            pl.BlockSpec((D, D), lambda b, l: (0, 0)),
            pl.BlockSpec((1, D), lambda b, l: (0, 0)),
            pl.BlockSpec((1, D), lambda b, l: (0, 0)),
            pl.BlockSpec((1, D), lambda b, l: (0, 0)),
            pl.BlockSpec((1, D), lambda b, l: (0, 0)),
            pl.BlockSpec((1, D), lambda b, l: (0, 0)),
            pl.BlockSpec((D, D), lambda b, l: (0, 0)),
            pl.BlockSpec((1, D), lambda b, l: (0, 0)),
            pl.BlockSpec((D, D), lambda b, l: (0, 0)),
            pl.BlockSpec((1, D), lambda b, l: (0, 0)),
        ],
        out_specs=pl.BlockSpec((1, LBC, D), lambda b, l: (b, l, 0)),
        out_shape=jax.ShapeDtypeStruct((B, L, D), jnp.float32),
    )(o5, lg5, h, wout, row(bout), row(g1), row(b1), row(g2), row(b2),
      wc1, row(bc1), wc2, row(bc2))


def _final_body(h_ref, g_ref, b_ref, wp_ref, bp_ref, out_ref):
    x = h_ref[0]
    m = jnp.mean(x, axis=-1, keepdims=True)
    v = jnp.mean((x - m) ** 2, axis=-1, keepdims=True)
    x = (x - m) / jnp.sqrt(v + 1e-5) * g_ref[...][0:1, :] + b_ref[...][0:1, :]
    bf = lambda t: t.astype(jnp.bfloat16).astype(jnp.float32)
    y = jnp.sum(bf(x) * bf(wp_ref[...][0:1, :]), axis=-1) + bp_ref[0, 0]
    out_ref[0, 0] = y


def _final(htail, lnf_g, lnf_b, wproj, bproj):
    out = pl.pallas_call(
        _final_body,
        grid=(B,),
        in_specs=[
            pl.BlockSpec((1, PRED, D), lambda b: (b, 0, 0)),
            pl.BlockSpec((1, D), lambda b: (0, 0)),
            pl.BlockSpec((1, D), lambda b: (0, 0)),
            pl.BlockSpec((1, D), lambda b: (0, 0)),
            pl.BlockSpec((1, 1), lambda b: (0, 0)),
        ],
        out_specs=pl.BlockSpec((1, 1, PRED), lambda b: (b, 0, 0)),
        out_shape=jax.ShapeDtypeStruct((B, 1, PRED), jnp.float32),
    )(htail, lnf_g.reshape(1, D), lnf_b.reshape(1, D),
      wproj.reshape(1, D), bproj.reshape(1, 1))
    return out[:, 0, :]


_NC, _NS = 2, 16
_NW = _NC * _NS
_NTASK = BH * NHASH
_TPW = _NTASK // _NW


def _sc_sort(bkts):
    mesh = plsc.VectorSubcoreMesh(core_axis_name="c", subcore_axis_name="s",
                                  num_cores=_NC, num_subcores=_NS)

    @functools.partial(
        pl.kernel,
        out_type=[jax.ShapeDtypeStruct((_NTASK, L), jnp.int32),
                  jax.ShapeDtypeStruct((_NTASK, L), jnp.int32)],
        mesh=mesh,
        scratch_types=[
            pltpu.VMEM((L,), jnp.int32),
            pltpu.VMEM((L,), jnp.int32),
            pltpu.VMEM((NBUCK * 16,), jnp.int32),
            pltpu.VMEM((NBUCK * 16,), jnp.int32),
            pltpu.VMEM((L,), jnp.int32),
            pltpu.VMEM((L,), jnp.int32),
        ],
        compiler_params=pltpu.CompilerParams(needs_layout_passes=False),
    )
    def k(b_hbm, st_hbm, sp_hbm, bvec, rank, cnt, base, stv, spv):
        wid = lax.axis_index("s") * _NC + lax.axis_index("c")
        lane = lax.iota(jnp.int32, 16)

        def task(ti, _):
            t = wid * _TPW + ti
            pltpu.sync_copy(b_hbm.at[t], bvec)

            def zero(i, _):
                cnt[pl.ds(i * 16, 16)] = jnp.zeros((16,), jnp.int32)
                return 0
            lax.fori_loop(0, NBUCK, zero, 0)

            def p1(i, _):
                pos = lane * 128 + i
                b16 = plsc.load_gather(bvec, [pos])
                ca = b16 * 16 + lane
                old = plsc.load_gather(cnt, [ca])
                plsc.store_scatter(cnt, [ca], old + 1)
                plsc.store_scatter(rank, [pos], old)
                return 0
            lax.fori_loop(0, 128, p1, 0)

            def p2(b, carry):
                c16 = cnt[pl.ds(b * 16, 16)]
                inc = plsc.cumsum(c16)
                base[pl.ds(b * 16, 16)] = inc - c16 + carry
                return carry + jnp.sum(c16)
            lax.fori_loop(0, NBUCK, p2, jnp.int32(0))

            def p3(i, _):
                pos = lane * 128 + i
                b16 = plsc.load_gather(bvec, [pos])
                rk = plsc.load_gather(rank, [pos])
                bs = plsc.load_gather(base, [b16 * 16 + lane])
                sp16 = bs + rk
                plsc.store_scatter(spv, [pos], sp16)
                plsc.store_scatter(stv, [sp16], pos)
                return 0
            lax.fori_loop(0, 128, p3, 0)

            pltpu.sync_copy(stv, st_hbm.at[t])
            pltpu.sync_copy(spv, sp_hbm.at[t])
            return 0

        lax.fori_loop(0, _TPW, task, 0)

    return k(bkts)


def _route(qk, v, bkts):
    qkh = qk.reshape(B, L, H, DH).transpose(0, 2, 1, 3).reshape(BH, L, DH)
    vh = v.reshape(B, L, H, DH).transpose(0, 2, 1, 3).reshape(BH, L, DH)
    st_t, sp_t = _sc_sort(bkts.reshape(_NTASK, L))
    st = st_t.reshape(BH, TOT)
    glob = sp_t.reshape(BH, NHASH, L) + (jnp.arange(NHASH, dtype=jnp.int32)
                                         * L)[None, :, None]
    glob = glob.reshape(BH, TOT)
    sqk = jnp.take_along_axis(qkh, st[..., None], axis=1)
    sv = jnp.take_along_axis(vh, st[..., None], axis=1)
    return sqk, sv, st, glob


def kernel(x_enc, Wemb, Wqk, Wv, Wout, bout, g1, b1, g2, b2, Wc1, bc1,
           Wc2, bc2, rotations, lnf_g, lnf_b, Wproj, bproj):
    pe = _positional_embedding()
    xp = jnp.concatenate([x_enc[:, -1:], x_enc, x_enc[:, :1]], axis=1)
    win = jnp.stack([xp[:, :-2], xp[:, 1:-1], xp[:, 2:]], axis=-1)
    w3 = jnp.zeros((8, D), jnp.float32).at[:3, :].set(Wemb[:, 0, :].T)
    h = _embed(win, pe, w3)

    for i in range(E):
        rot = rotations[i].reshape(DH, NHASH * (NBUCK // 2))
        qk, v, bkts = _proj(h, Wqk[i], Wv[i], rot)
        sqk, sv, st, glob = _route(qk, v, bkts)
        st3 = st.reshape(BH, NCH, BUCKET)
        so, slog = _attn(sqk, sv, st3)
        slog = slog.reshape(BH, TOT)
        o = jnp.take_along_axis(so, glob[..., None], axis=1)
        lg = jnp.take_along_axis(slog, glob, axis=1)
        o5 = o.reshape(B, H, NHASH, L, DH)
        lg5 = lg.reshape(B, H, NHASH, L)
        h = _combine(o5, lg5, h, Wout[i], bout[i], g1[i], b1[i], g2[i],
                     b2[i], Wc1[i], bc1[i], Wc2[i], bc2[i])

    htail = h[:, -PRED:, :]
    return _final(htail, lnf_g, lnf_b, Wproj, bproj)
